# baseline (speedup 1.0000x reference)
import functools

import jax
import jax.numpy as jnp
from jax import lax
from jax.experimental import pallas as pl
from jax.experimental.pallas import tpu as pltpu
from jax.experimental.pallas import tpu_sc as plsc

H = 128
HW = H // 2
N = 10000
E = 320000
NT = 4

NC, NS = 2, 16
NW = NC * NS
CK = 80
NSPLIT = 5
ES = E // NSPLIT
EWS = ES // NW
CKG = 40
NCHG = EWS // CKG
NCHS = ES // CK // NW
NP = 10240
RPT = NP // NS
SRB = 64

NB = 1000
NB1 = 2000
BE = 1280


def _pq_body(x_ref, w_ref, b_ref, o_ref):
    o_ref[0] = (
        jnp.dot(x_ref[...], w_ref[0], preferred_element_type=jnp.float32)
        + b_ref[0]
    )


def _msgs_body(p_ref, q_ref, m_ref, w2c_ref, b2p_ref, o_ref):
    hb = jnp.maximum(p_ref[...] + q_ref[...], 0.0).astype(jnp.bfloat16)
    mm = m_ref[...]
    allm = lax.dot_general(
        hb, w2c_ref[...], (((1,), (0,)), ((), ())),
        preferred_element_type=jnp.float32,
    )
    acc = jnp.dot(mm, b2p_ref[...], preferred_element_type=jnp.float32)
    for t in range(NT):
        acc = acc + mm[:, t:t + 1] * allm[:, t * H:(t + 1) * H]
    o_ref[...] = acc


def _gru_body(p0_ref, p1_ref, x_ref, wih_ref, whh_ref, bih_ref, bhh_ref,
              o_ref):
    x = x_ref[...]
    agg = p0_ref[0] + p1_ref[0]
    gi = lax.dot_general(
        agg, wih_ref[...], (((1,), (1,)), ((), ())),
        preferred_element_type=jnp.float32,
    ) + bih_ref[0]
    gh = lax.dot_general(
        x, whh_ref[...], (((1,), (1,)), ((), ())),
        preferred_element_type=jnp.float32,
    ) + bhh_ref[0]
    r = jax.nn.sigmoid(gi[:, :H] + gh[:, :H])
    z = jax.nn.sigmoid(gi[:, H:2 * H] + gh[:, H:2 * H])
    n = jnp.tanh(gi[:, 2 * H:] + r * gh[:, 2 * H:])
    o_ref[...] = (1.0 - z) * n + z * x


_MESH = plsc.VectorSubcoreMesh(
    core_axis_name="c", subcore_axis_name="s", num_cores=NC, num_subcores=NS)


def _gather_body(tab_hbm, src_hbm, dst_hbm, et_hbm, p_hbm, q_hbm,
                 src_v, dst_v, et_v, idxp_v, idxq_v,
                 p0_v, p1_v, q0_v, q1_v,
                 semp0, semp1, semq0, semq1, semw):
    c = lax.axis_index("c")
    s = lax.axis_index("s")
    wid = c * NS + s
    ebase = wid * EWS
    pltpu.sync_copy(src_hbm.at[pl.ds(ebase, EWS)], src_v)
    pltpu.sync_copy(dst_hbm.at[pl.ds(ebase, EWS)], dst_v)
    pltpu.sync_copy(et_hbm.at[pl.ds(ebase, EWS)], et_v)

    def idx_body(j, carry):
        sl = pl.ds(j * 16, 16)
        t16 = et_v[sl] * N
        idxp_v[sl] = t16 + src_v[sl]
        idxq_v[sl] = (NT * N) + t16 + dst_v[sl]
        return carry

    lax.fori_loop(0, EWS // 16, idx_body, 0)

    pbuf = (p0_v, p1_v)
    qbuf = (q0_v, q1_v)
    psem = (semp0, semp1)
    qsem = (semq0, semq1)
    cps = [None, None]
    cqs = [None, None]
    wrs = [[], []]

    def issue(i, par):
        cb = i * CKG
        cps[par] = pltpu.async_copy(
            tab_hbm.at[idxp_v.at[pl.ds(cb, CKG)]], pbuf[par], psem[par])
        cqs[par] = pltpu.async_copy(
            tab_hbm.at[idxq_v.at[pl.ds(cb, CKG)]], qbuf[par], qsem[par])

    issue(0, 0)
    for i in range(NCHG):
        par = i & 1
        cps[par].wait()
        cqs[par].wait()
        if i + 1 < NCHG:
            npar = (i + 1) & 1
            for w in wrs[npar]:
                w.wait()
            wrs[npar] = []
            issue(i + 1, npar)
        cb = ebase + i * CKG
        wrs[par] = [
            pltpu.async_copy(pbuf[par], p_hbm.at[pl.ds(cb, CKG)], semw),
            pltpu.async_copy(qbuf[par], q_hbm.at[pl.ds(cb, CKG)], semw),
        ]
    for ws in wrs:
        for w in ws:
            w.wait()


def _scatter_body(m0_hbm, m1_hbm, m2_hbm, m3_hbm, m4_hbm, dst5_hbm, out_hbm,
                  dst_v, m0_v, m1_v, stage_v, acc_sh, sem0, sem1):
    c = lax.axis_index("c")
    s = lax.axis_index("s")
    wid = c * NS + s

    def zero_body(r, carry):
        for cc in range(H // 16):
            stage_v[r, pl.ds(cc * 16, 16)] = jnp.zeros((16,), jnp.float32)
        return carry

    lax.fori_loop(0, SRB, zero_body, 0)
    for k in range(RPT // SRB):
        pltpu.sync_copy(stage_v, acc_sh.at[pl.ds(s * RPT + k * SRB, SRB)])
    plsc.subcore_barrier()

    bufs = (m0_v, m1_v)
    sems = (sem0, sem1)
    cps = [None, None]
    for k, mk in enumerate((m0_hbm, m1_hbm, m2_hbm, m3_hbm, m4_hbm)):
        pltpu.sync_copy(dst5_hbm.at[k, wid], dst_v)
        cps[0] = pltpu.async_copy(mk.at[pl.ds(wid * CK, CK)], m0_v, sem0)
        for i in range(NCHS):
            p = i & 1
            cps[p].wait()
            if i + 1 < NCHS:
                q = (i + 1) & 1
                cps[q] = pltpu.async_copy(
                    mk.at[pl.ds((wid + (i + 1) * NW) * CK, CK)],
                    bufs[q], sems[q])
            pltpu.sync_copy(bufs[p], acc_sh.at[dst_v.at[i]], add=True)
    plsc.subcore_barrier()
    for k in range(RPT // SRB):
        pltpu.sync_copy(acc_sh.at[pl.ds(s * RPT + k * SRB, SRB)], stage_v)
        pltpu.sync_copy(stage_v, out_hbm.at[c, pl.ds(s * RPT + k * SRB, SRB)])


_gather_kernel = functools.partial(
    pl.kernel,
    out_type=(
        jax.ShapeDtypeStruct((ES, H), jnp.float32),
        jax.ShapeDtypeStruct((ES, H), jnp.float32),
    ),
    mesh=_MESH,
    scratch_types=[
        pltpu.VMEM((EWS,), jnp.int32),
        pltpu.VMEM((EWS,), jnp.int32),
        pltpu.VMEM((EWS,), jnp.int32),
        pltpu.VMEM((EWS,), jnp.int32),
        pltpu.VMEM((EWS,), jnp.int32),
        pltpu.VMEM((CKG, H), jnp.float32),
        pltpu.VMEM((CKG, H), jnp.float32),
        pltpu.VMEM((CKG, H), jnp.float32),
        pltpu.VMEM((CKG, H), jnp.float32),
        pltpu.SemaphoreType.DMA,
        pltpu.SemaphoreType.DMA,
        pltpu.SemaphoreType.DMA,
        pltpu.SemaphoreType.DMA,
        pltpu.SemaphoreType.DMA,
    ],
)(_gather_body)


_scatter_kernel = functools.partial(
    pl.kernel,
    out_type=jax.ShapeDtypeStruct((NC, NP, H), jnp.float32),
    mesh=_MESH,
    scratch_types=[
        pltpu.VMEM((NCHS, CK), jnp.int32),
        pltpu.VMEM((CK, H), jnp.float32),
        pltpu.VMEM((CK, H), jnp.float32),
        pltpu.VMEM((SRB, H), jnp.float32),
        pltpu.VMEM_SHARED((NP, H), jnp.float32),
        pltpu.SemaphoreType.DMA,
        pltpu.SemaphoreType.DMA,
    ],
)(_scatter_body)


def kernel(x, edge_index, edge_type, W1, b1, W2, b2, W_ih, W_hh, b_ih, b_hh):
    src = edge_index[0]
    dst = edge_index[1]
    et = edge_type.astype(jnp.int32)

    w_p = jnp.transpose(W1[:, :, :H], (0, 2, 1))
    w_q = jnp.transpose(W1[:, :, H:], (0, 2, 1))
    wstack = jnp.concatenate([w_p, w_q], axis=0)
    bstack = jnp.concatenate(
        [jnp.zeros((NT, H), jnp.float32), b1], axis=0).reshape(2 * NT, 1, H)

    pq = pl.pallas_call(
        _pq_body,
        grid=(2 * NT, N // NB1),
        in_specs=[
            pl.BlockSpec((NB1, H), lambda j, n: (n, 0)),
            pl.BlockSpec((1, H, H), lambda j, n: (j, 0, 0)),
            pl.BlockSpec((1, 1, H), lambda j, n: (j, 0, 0)),
        ],
        out_specs=pl.BlockSpec((1, NB1, H), lambda j, n: (j, n, 0)),
        out_shape=jax.ShapeDtypeStruct((2 * NT, N, H), jnp.float32),
    )(x, wstack, bstack)
    table = pq.reshape(2 * NT * N, H)

    w2c = jnp.transpose(W2, (2, 0, 1)).reshape(H, NT * H)
    w2c = w2c.astype(jnp.bfloat16)
    mask8 = (et[:, None] == jnp.arange(8, dtype=jnp.int32)[None, :])
    mask8 = mask8.astype(jnp.float32)
    b2pad = jnp.concatenate([b2, jnp.zeros((4, H), jnp.float32)], axis=0)

    msgs_splits = []
    for k in range(NSPLIT):
        sl = slice(k * ES, (k + 1) * ES)
        p_rows, q_rows = _gather_kernel(table, src[sl], dst[sl], et[sl])
        msgs_k = pl.pallas_call(
            _msgs_body,
            grid=(ES // BE,),
            in_specs=[
                pl.BlockSpec((BE, H), lambda i: (i, 0)),
                pl.BlockSpec((BE, H), lambda i: (i, 0)),
                pl.BlockSpec((BE, 8), lambda i: (i, 0)),
                pl.BlockSpec((H, NT * H), lambda i: (0, 0)),
                pl.BlockSpec((8, H), lambda i: (0, 0)),
            ],
            out_specs=pl.BlockSpec((BE, H), lambda i: (i, 0)),
            out_shape=jax.ShapeDtypeStruct((ES, H), jnp.float32),
        )(p_rows, q_rows, mask8[sl], w2c, b2pad)
        msgs_splits.append(msgs_k)

    dst5 = dst.reshape(NSPLIT, NCHS, NW, CK).transpose(0, 2, 1, 3)
    partials = _scatter_kernel(*msgs_splits, dst5)

    x_new = pl.pallas_call(
        _gru_body,
        grid=(N // NB,),
        in_specs=[
            pl.BlockSpec((1, NB, H), lambda n: (0, n, 0)),
            pl.BlockSpec((1, NB, H), lambda n: (1, n, 0)),
            pl.BlockSpec((NB, H), lambda n: (n, 0)),
            pl.BlockSpec((3 * H, H), lambda n: (0, 0)),
            pl.BlockSpec((3 * H, H), lambda n: (0, 0)),
            pl.BlockSpec((1, 3 * H), lambda n: (0, 0)),
            pl.BlockSpec((1, 3 * H), lambda n: (0, 0)),
        ],
        out_specs=pl.BlockSpec((NB, H), lambda n: (n, 0)),
        out_shape=jax.ShapeDtypeStruct((N, H), jnp.float32),
    )(partials, partials, x, W_ih, W_hh, b_ih.reshape(1, 3 * H),
      b_hh.reshape(1, 3 * H))
    return x_new


# speedup vs baseline: 1.3005x; 1.3005x over previous
import functools

import jax
import jax.numpy as jnp
from jax import lax
from jax.experimental import pallas as pl
from jax.experimental.pallas import tpu as pltpu
from jax.experimental.pallas import tpu_sc as plsc

H = 128
HW = H // 2
N = 10000
E = 320000
NT = 4

NC, NS = 2, 16
NW = NC * NS
CK = 80
NSPLIT = 5
ES = E // NSPLIT
EWS = ES // NW
CKG = 40
NCHG = EWS // CKG
NCHS = ES // CK // NW
NP = 10240
RPT = NP // NS
SRB = 64

NB = 1000
NB1 = 2000
BE = 1280


def _pq_body(x_ref, w_ref, b_ref, o_ref):
    o_ref[0] = (
        jnp.dot(x_ref[...], w_ref[0], preferred_element_type=jnp.float32)
        + b_ref[0]
    )


def _msgs_body(p_ref, q_ref, et_ref, w2c_ref, b2_ref, o_ref):
    hb = jnp.maximum(p_ref[...] + q_ref[...], 0.0).astype(jnp.bfloat16)
    et = et_ref[...].reshape(BE, 1)
    allm = lax.dot_general(
        hb, w2c_ref[...], (((1,), (0,)), ((), ())),
        preferred_element_type=jnp.float32,
    )
    acc = jnp.zeros((BE, H), jnp.float32)
    for t in range(NT):
        fm = (et == t).astype(jnp.float32)
        acc = acc + fm * (allm[:, t * H:(t + 1) * H] + b2_ref[t])
    o_ref[...] = acc


def _gru_body(p0_ref, p1_ref, x_ref, wih_ref, whh_ref, bih_ref, bhh_ref,
              o_ref):
    x = x_ref[...]
    agg = p0_ref[0] + p1_ref[0]
    gi = lax.dot_general(
        agg, wih_ref[...], (((1,), (1,)), ((), ())),
        preferred_element_type=jnp.float32,
    ) + bih_ref[0]
    gh = lax.dot_general(
        x, whh_ref[...], (((1,), (1,)), ((), ())),
        preferred_element_type=jnp.float32,
    ) + bhh_ref[0]
    r = jax.nn.sigmoid(gi[:, :H] + gh[:, :H])
    z = jax.nn.sigmoid(gi[:, H:2 * H] + gh[:, H:2 * H])
    n = jnp.tanh(gi[:, 2 * H:] + r * gh[:, 2 * H:])
    o_ref[...] = (1.0 - z) * n + z * x


_MESH = plsc.VectorSubcoreMesh(
    core_axis_name="c", subcore_axis_name="s", num_cores=NC, num_subcores=NS)


def _gather_body(tab_hbm, src_hbm, dst_hbm, et_hbm, p_hbm, q_hbm,
                 src_v, dst_v, et_v, idxp_v, idxq_v,
                 p0_v, p1_v, q0_v, q1_v,
                 semp0, semp1, semq0, semq1, semw):
    c = lax.axis_index("c")
    s = lax.axis_index("s")
    wid = c * NS + s
    ebase = wid * EWS
    pltpu.sync_copy(src_hbm.at[pl.ds(ebase, EWS)], src_v)
    pltpu.sync_copy(dst_hbm.at[pl.ds(ebase, EWS)], dst_v)
    pltpu.sync_copy(et_hbm.at[pl.ds(ebase, EWS)], et_v)

    def idx_body(j, carry):
        sl = pl.ds(j * 16, 16)
        t16 = et_v[sl] * N
        idxp_v[sl] = t16 + src_v[sl]
        idxq_v[sl] = (NT * N) + t16 + dst_v[sl]
        return carry

    lax.fori_loop(0, EWS // 16, idx_body, 0)

    pbuf = (p0_v, p1_v)
    qbuf = (q0_v, q1_v)
    psem = (semp0, semp1)
    qsem = (semq0, semq1)
    cps = [None, None]
    cqs = [None, None]
    wrs = [[], []]

    def issue(i, par):
        cb = i * CKG
        cps[par] = pltpu.async_copy(
            tab_hbm.at[idxp_v.at[pl.ds(cb, CKG)]], pbuf[par], psem[par])
        cqs[par] = pltpu.async_copy(
            tab_hbm.at[idxq_v.at[pl.ds(cb, CKG)]], qbuf[par], qsem[par])

    issue(0, 0)
    for i in range(NCHG):
        par = i & 1
        cps[par].wait()
        cqs[par].wait()
        if i + 1 < NCHG:
            npar = (i + 1) & 1
            for w in wrs[npar]:
                w.wait()
            wrs[npar] = []
            issue(i + 1, npar)
        cb = ebase + i * CKG
        wrs[par] = [
            pltpu.async_copy(pbuf[par], p_hbm.at[pl.ds(cb, CKG)], semw),
            pltpu.async_copy(qbuf[par], q_hbm.at[pl.ds(cb, CKG)], semw),
        ]
    for ws in wrs:
        for w in ws:
            w.wait()


def _scatter_body(m0_hbm, m1_hbm, m2_hbm, m3_hbm, m4_hbm, dst5_hbm, out_hbm,
                  dst_v, m0_v, m1_v, stage_v, acc_sh, sem0, sem1):
    c = lax.axis_index("c")
    s = lax.axis_index("s")
    wid = c * NS + s

    def zero_body(r, carry):
        for cc in range(H // 16):
            stage_v[r, pl.ds(cc * 16, 16)] = jnp.zeros((16,), jnp.float32)
        return carry

    lax.fori_loop(0, SRB, zero_body, 0)
    for k in range(RPT // SRB):
        pltpu.sync_copy(stage_v, acc_sh.at[pl.ds(s * RPT + k * SRB, SRB)])
    plsc.subcore_barrier()

    bufs = (m0_v, m1_v)
    sems = (sem0, sem1)
    cps = [None, None]
    for k, mk in enumerate((m0_hbm, m1_hbm, m2_hbm, m3_hbm, m4_hbm)):
        pltpu.sync_copy(dst5_hbm.at[k, wid], dst_v)
        cps[0] = pltpu.async_copy(mk.at[pl.ds(wid * CK, CK)], m0_v, sem0)
        for i in range(NCHS):
            p = i & 1
            cps[p].wait()
            if i + 1 < NCHS:
                q = (i + 1) & 1
                cps[q] = pltpu.async_copy(
                    mk.at[pl.ds((wid + (i + 1) * NW) * CK, CK)],
                    bufs[q], sems[q])
            pltpu.sync_copy(bufs[p], acc_sh.at[dst_v.at[i]], add=True)
    plsc.subcore_barrier()
    for k in range(RPT // SRB):
        pltpu.sync_copy(acc_sh.at[pl.ds(s * RPT + k * SRB, SRB)], stage_v)
        pltpu.sync_copy(stage_v, out_hbm.at[c, pl.ds(s * RPT + k * SRB, SRB)])


_gather_kernel = functools.partial(
    pl.kernel,
    out_type=(
        jax.ShapeDtypeStruct((ES, H), jnp.float32),
        jax.ShapeDtypeStruct((ES, H), jnp.float32),
    ),
    mesh=_MESH,
    scratch_types=[
        pltpu.VMEM((EWS,), jnp.int32),
        pltpu.VMEM((EWS,), jnp.int32),
        pltpu.VMEM((EWS,), jnp.int32),
        pltpu.VMEM((EWS,), jnp.int32),
        pltpu.VMEM((EWS,), jnp.int32),
        pltpu.VMEM((CKG, H), jnp.float32),
        pltpu.VMEM((CKG, H), jnp.float32),
        pltpu.VMEM((CKG, H), jnp.float32),
        pltpu.VMEM((CKG, H), jnp.float32),
        pltpu.SemaphoreType.DMA,
        pltpu.SemaphoreType.DMA,
        pltpu.SemaphoreType.DMA,
        pltpu.SemaphoreType.DMA,
        pltpu.SemaphoreType.DMA,
    ],
)(_gather_body)


_scatter_kernel = functools.partial(
    pl.kernel,
    out_type=jax.ShapeDtypeStruct((NC, NP, H), jnp.float32),
    mesh=_MESH,
    scratch_types=[
        pltpu.VMEM((NCHS, CK), jnp.int32),
        pltpu.VMEM((CK, H), jnp.float32),
        pltpu.VMEM((CK, H), jnp.float32),
        pltpu.VMEM((SRB, H), jnp.float32),
        pltpu.VMEM_SHARED((NP, H), jnp.float32),
        pltpu.SemaphoreType.DMA,
        pltpu.SemaphoreType.DMA,
    ],
)(_scatter_body)


def kernel(x, edge_index, edge_type, W1, b1, W2, b2, W_ih, W_hh, b_ih, b_hh):
    src = edge_index[0]
    dst = edge_index[1]
    et = edge_type.astype(jnp.int32)

    w_p = jnp.transpose(W1[:, :, :H], (0, 2, 1))
    w_q = jnp.transpose(W1[:, :, H:], (0, 2, 1))
    wstack = jnp.concatenate([w_p, w_q], axis=0)
    bstack = jnp.concatenate(
        [jnp.zeros((NT, H), jnp.float32), b1], axis=0).reshape(2 * NT, 1, H)

    pq = pl.pallas_call(
        _pq_body,
        grid=(2 * NT, N // NB1),
        in_specs=[
            pl.BlockSpec((NB1, H), lambda j, n: (n, 0)),
            pl.BlockSpec((1, H, H), lambda j, n: (j, 0, 0)),
            pl.BlockSpec((1, 1, H), lambda j, n: (j, 0, 0)),
        ],
        out_specs=pl.BlockSpec((1, NB1, H), lambda j, n: (j, n, 0)),
        out_shape=jax.ShapeDtypeStruct((2 * NT, N, H), jnp.float32),
    )(x, wstack, bstack)
    table = pq.reshape(2 * NT * N, H)

    w2c = jnp.transpose(W2, (2, 0, 1)).reshape(H, NT * H)
    w2c = w2c.astype(jnp.bfloat16)

    msgs_splits = []
    for k in range(NSPLIT):
        sl = slice(k * ES, (k + 1) * ES)
        p_rows, q_rows = _gather_kernel(table, src[sl], dst[sl], et[sl])
        msgs_k = pl.pallas_call(
            _msgs_body,
            grid=(ES // BE,),
            in_specs=[
                pl.BlockSpec((BE, H), lambda i: (i, 0)),
                pl.BlockSpec((BE, H), lambda i: (i, 0)),
                pl.BlockSpec((1, 1, BE), lambda i: (i, 0, 0)),
                pl.BlockSpec((H, NT * H), lambda i: (0, 0)),
                pl.BlockSpec((NT, H), lambda i: (0, 0)),
            ],
            out_specs=pl.BlockSpec((BE, H), lambda i: (i, 0)),
            out_shape=jax.ShapeDtypeStruct((ES, H), jnp.float32),
        )(p_rows, q_rows, et[sl].reshape(ES // BE, 1, BE), w2c, b2)
        msgs_splits.append(msgs_k)

    dst5 = dst.reshape(NSPLIT, NCHS, NW, CK).transpose(0, 2, 1, 3)
    partials = _scatter_kernel(*msgs_splits, dst5)

    x_new = pl.pallas_call(
        _gru_body,
        grid=(N // NB,),
        in_specs=[
            pl.BlockSpec((1, NB, H), lambda n: (0, n, 0)),
            pl.BlockSpec((1, NB, H), lambda n: (1, n, 0)),
            pl.BlockSpec((NB, H), lambda n: (n, 0)),
            pl.BlockSpec((3 * H, H), lambda n: (0, 0)),
            pl.BlockSpec((3 * H, H), lambda n: (0, 0)),
            pl.BlockSpec((1, 3 * H), lambda n: (0, 0)),
            pl.BlockSpec((1, 3 * H), lambda n: (0, 0)),
        ],
        out_specs=pl.BlockSpec((NB, H), lambda n: (n, 0)),
        out_shape=jax.ShapeDtypeStruct((N, H), jnp.float32),
    )(partials, partials, x, W_ih, W_hh, b_ih.reshape(1, 3 * H),
      b_hh.reshape(1, 3 * H))
    return x_new


# speedup vs baseline: 1.3568x; 1.0433x over previous
import functools

import jax
import jax.numpy as jnp
from jax import lax
from jax.experimental import pallas as pl
from jax.experimental.pallas import tpu as pltpu
from jax.experimental.pallas import tpu_sc as plsc

H = 128
HW = H // 2
N = 10000
E = 320000
NT = 4

NC, NS = 2, 16
NW = NC * NS
CK = 80
NSPLIT = 5
ES = E // NSPLIT
EWS = ES // NW
CKG = 40
NCHG = EWS // CKG
NCHS = ES // CK // NW
NP = 10240
RPT = NP // NS
SRB = 64

NB = 1000
NB1 = 2000
BE = 1280


def _pq_body(x_ref, w_ref, b_ref, o_ref):
    o_ref[0] = (
        jnp.dot(x_ref[...], w_ref[0], preferred_element_type=jnp.float32)
        + b_ref[0]
    )


def _msgs_body(p_ref, q_ref, et_ref, w2c_ref, b2_ref, o_ref):
    hb = jnp.maximum(p_ref[...] + q_ref[...], 0.0).astype(jnp.bfloat16)
    et = et_ref[...].reshape(BE, 1)
    allm = lax.dot_general(
        hb, w2c_ref[...], (((1,), (0,)), ((), ())),
        preferred_element_type=jnp.float32,
    )
    acc = jnp.zeros((BE, H), jnp.float32)
    for t in range(NT):
        fm = (et == t).astype(jnp.float32)
        acc = acc + fm * (allm[:, t * H:(t + 1) * H] + b2_ref[t])
    o_ref[...] = acc


def _gru_body(p0_ref, p1_ref, p2_ref, p3_ref, x_ref, wih_ref, whh_ref,
              bih_ref, bhh_ref, o_ref):
    x = x_ref[...]
    agg = (p0_ref[0] + p1_ref[0]) + (p2_ref[0] + p3_ref[0])
    gi = lax.dot_general(
        agg, wih_ref[...], (((1,), (1,)), ((), ())),
        preferred_element_type=jnp.float32,
    ) + bih_ref[0]
    gh = lax.dot_general(
        x, whh_ref[...], (((1,), (1,)), ((), ())),
        preferred_element_type=jnp.float32,
    ) + bhh_ref[0]
    r = jax.nn.sigmoid(gi[:, :H] + gh[:, :H])
    z = jax.nn.sigmoid(gi[:, H:2 * H] + gh[:, H:2 * H])
    n = jnp.tanh(gi[:, 2 * H:] + r * gh[:, 2 * H:])
    o_ref[...] = (1.0 - z) * n + z * x


_MESH = plsc.VectorSubcoreMesh(
    core_axis_name="c", subcore_axis_name="s", num_cores=NC, num_subcores=NS)


def _gather_body(tab_hbm, src_hbm, dst_hbm, et_hbm, p_hbm, q_hbm,
                 src_v, dst_v, et_v, idxp_v, idxq_v,
                 p0_v, p1_v, q0_v, q1_v,
                 semp0, semp1, semq0, semq1, semw):
    c = lax.axis_index("c")
    s = lax.axis_index("s")
    wid = c * NS + s
    ebase = wid * EWS
    pltpu.sync_copy(src_hbm.at[pl.ds(ebase, EWS)], src_v)
    pltpu.sync_copy(dst_hbm.at[pl.ds(ebase, EWS)], dst_v)
    pltpu.sync_copy(et_hbm.at[pl.ds(ebase, EWS)], et_v)

    def idx_body(j, carry):
        sl = pl.ds(j * 16, 16)
        t16 = et_v[sl] * N
        idxp_v[sl] = t16 + src_v[sl]
        idxq_v[sl] = (NT * N) + t16 + dst_v[sl]
        return carry

    lax.fori_loop(0, EWS // 16, idx_body, 0)

    pbuf = (p0_v, p1_v)
    qbuf = (q0_v, q1_v)
    psem = (semp0, semp1)
    qsem = (semq0, semq1)
    cps = [None, None]
    cqs = [None, None]
    wrs = [[], []]

    def issue(i, par):
        cb = i * CKG
        cps[par] = pltpu.async_copy(
            tab_hbm.at[idxp_v.at[pl.ds(cb, CKG)]], pbuf[par], psem[par])
        cqs[par] = pltpu.async_copy(
            tab_hbm.at[idxq_v.at[pl.ds(cb, CKG)]], qbuf[par], qsem[par])

    issue(0, 0)
    for i in range(NCHG):
        par = i & 1
        cps[par].wait()
        cqs[par].wait()
        if i + 1 < NCHG:
            npar = (i + 1) & 1
            for w in wrs[npar]:
                w.wait()
            wrs[npar] = []
            issue(i + 1, npar)
        cb = ebase + i * CKG
        wrs[par] = [
            pltpu.async_copy(pbuf[par], p_hbm.at[pl.ds(cb, CKG)], semw),
            pltpu.async_copy(qbuf[par], q_hbm.at[pl.ds(cb, CKG)], semw),
        ]
    for ws in wrs:
        for w in ws:
            w.wait()


def _make_scatter_body(split_ids):
    nm = len(split_ids)

    def body(*args):
        ms = args[:nm]
        (dst5_hbm, out_hbm, dst_v, m0_v, m1_v, stage_v, acc_sh,
         sem0, sem1) = args[nm:]
        c = lax.axis_index("c")
        s = lax.axis_index("s")
        wid = c * NS + s

        def zero_body(r, carry):
            for cc in range(H // 16):
                stage_v[r, pl.ds(cc * 16, 16)] = jnp.zeros((16,),
                                                           jnp.float32)
            return carry

        lax.fori_loop(0, SRB, zero_body, 0)
        for k in range(RPT // SRB):
            pltpu.sync_copy(stage_v, acc_sh.at[pl.ds(s * RPT + k * SRB, SRB)])
        plsc.subcore_barrier()

        bufs = (m0_v, m1_v)
        sems = (sem0, sem1)
        cps = [None, None]
        for k, mk in zip(split_ids, ms):
            pltpu.sync_copy(dst5_hbm.at[k, wid], dst_v)
            cps[0] = pltpu.async_copy(mk.at[pl.ds(wid * CK, CK)], m0_v, sem0)
            for i in range(NCHS):
                p = i & 1
                cps[p].wait()
                if i + 1 < NCHS:
                    q = (i + 1) & 1
                    cps[q] = pltpu.async_copy(
                        mk.at[pl.ds((wid + (i + 1) * NW) * CK, CK)],
                        bufs[q], sems[q])
                pltpu.sync_copy(bufs[p], acc_sh.at[dst_v.at[i]], add=True)
        plsc.subcore_barrier()
        for k in range(RPT // SRB):
            pltpu.sync_copy(acc_sh.at[pl.ds(s * RPT + k * SRB, SRB)], stage_v)
            pltpu.sync_copy(stage_v,
                            out_hbm.at[c, pl.ds(s * RPT + k * SRB, SRB)])

    return body


_gather_kernel = functools.partial(
    pl.kernel,
    out_type=(
        jax.ShapeDtypeStruct((ES, H), jnp.float32),
        jax.ShapeDtypeStruct((ES, H), jnp.float32),
    ),
    mesh=_MESH,
    scratch_types=[
        pltpu.VMEM((EWS,), jnp.int32),
        pltpu.VMEM((EWS,), jnp.int32),
        pltpu.VMEM((EWS,), jnp.int32),
        pltpu.VMEM((EWS,), jnp.int32),
        pltpu.VMEM((EWS,), jnp.int32),
        pltpu.VMEM((CKG, H), jnp.float32),
        pltpu.VMEM((CKG, H), jnp.float32),
        pltpu.VMEM((CKG, H), jnp.float32),
        pltpu.VMEM((CKG, H), jnp.float32),
        pltpu.SemaphoreType.DMA,
        pltpu.SemaphoreType.DMA,
        pltpu.SemaphoreType.DMA,
        pltpu.SemaphoreType.DMA,
        pltpu.SemaphoreType.DMA,
    ],
)(_gather_body)


def _make_scatter_kernel(split_ids):
    return functools.partial(
        pl.kernel,
        out_type=jax.ShapeDtypeStruct((NC, NP, H), jnp.float32),
        mesh=_MESH,
        scratch_types=[
            pltpu.VMEM((NCHS, CK), jnp.int32),
            pltpu.VMEM((CK, H), jnp.float32),
            pltpu.VMEM((CK, H), jnp.float32),
            pltpu.VMEM((SRB, H), jnp.float32),
            pltpu.VMEM_SHARED((NP, H), jnp.float32),
            pltpu.SemaphoreType.DMA,
            pltpu.SemaphoreType.DMA,
        ],
    )(_make_scatter_body(split_ids))


_scatter_kernel_a = _make_scatter_kernel((0, 1, 2))
_scatter_kernel_b = _make_scatter_kernel((3, 4))


def kernel(x, edge_index, edge_type, W1, b1, W2, b2, W_ih, W_hh, b_ih, b_hh):
    src = edge_index[0]
    dst = edge_index[1]
    et = edge_type.astype(jnp.int32)

    w_p = jnp.transpose(W1[:, :, :H], (0, 2, 1))
    w_q = jnp.transpose(W1[:, :, H:], (0, 2, 1))
    wstack = jnp.concatenate([w_p, w_q], axis=0)
    bstack = jnp.concatenate(
        [jnp.zeros((NT, H), jnp.float32), b1], axis=0).reshape(2 * NT, 1, H)

    pq = pl.pallas_call(
        _pq_body,
        grid=(2 * NT, N // NB1),
        in_specs=[
            pl.BlockSpec((NB1, H), lambda j, n: (n, 0)),
            pl.BlockSpec((1, H, H), lambda j, n: (j, 0, 0)),
            pl.BlockSpec((1, 1, H), lambda j, n: (j, 0, 0)),
        ],
        out_specs=pl.BlockSpec((1, NB1, H), lambda j, n: (j, n, 0)),
        out_shape=jax.ShapeDtypeStruct((2 * NT, N, H), jnp.float32),
    )(x, wstack, bstack)
    table = pq.reshape(2 * NT * N, H)

    w2c = jnp.transpose(W2, (2, 0, 1)).reshape(H, NT * H)
    w2c = w2c.astype(jnp.bfloat16)

    msgs_splits = []
    for k in range(NSPLIT):
        sl = slice(k * ES, (k + 1) * ES)
        p_rows, q_rows = _gather_kernel(table, src[sl], dst[sl], et[sl])
        msgs_k = pl.pallas_call(
            _msgs_body,
            grid=(ES // BE,),
            in_specs=[
                pl.BlockSpec((BE, H), lambda i: (i, 0)),
                pl.BlockSpec((BE, H), lambda i: (i, 0)),
                pl.BlockSpec((1, 1, BE), lambda i: (i, 0, 0)),
                pl.BlockSpec((H, NT * H), lambda i: (0, 0)),
                pl.BlockSpec((NT, H), lambda i: (0, 0)),
            ],
            out_specs=pl.BlockSpec((BE, H), lambda i: (i, 0)),
            out_shape=jax.ShapeDtypeStruct((ES, H), jnp.float32),
        )(p_rows, q_rows, et[sl].reshape(ES // BE, 1, BE), w2c, b2)
        msgs_splits.append(msgs_k)

    dst5 = dst.reshape(NSPLIT, NCHS, NW, CK).transpose(0, 2, 1, 3)
    partials_a = _scatter_kernel_a(*msgs_splits[:3], dst5)
    partials_b = _scatter_kernel_b(*msgs_splits[3:], dst5)

    x_new = pl.pallas_call(
        _gru_body,
        grid=(N // NB,),
        in_specs=[
            pl.BlockSpec((1, NB, H), lambda n: (0, n, 0)),
            pl.BlockSpec((1, NB, H), lambda n: (1, n, 0)),
            pl.BlockSpec((1, NB, H), lambda n: (0, n, 0)),
            pl.BlockSpec((1, NB, H), lambda n: (1, n, 0)),
            pl.BlockSpec((NB, H), lambda n: (n, 0)),
            pl.BlockSpec((3 * H, H), lambda n: (0, 0)),
            pl.BlockSpec((3 * H, H), lambda n: (0, 0)),
            pl.BlockSpec((1, 3 * H), lambda n: (0, 0)),
            pl.BlockSpec((1, 3 * H), lambda n: (0, 0)),
        ],
        out_specs=pl.BlockSpec((NB, H), lambda n: (n, 0)),
        out_shape=jax.ShapeDtypeStruct((N, H), jnp.float32),
    )(partials_a, partials_a, partials_b, partials_b, x, W_ih, W_hh,
      b_ih.reshape(1, 3 * H), b_hh.reshape(1, 3 * H))
    return x_new
